# baseline (speedup 1.0000x reference)
import functools

import numpy as np
import jax
import jax.numpy as jnp
from jax import lax
from jax.experimental import pallas as pl
from jax.experimental.pallas import tpu as pltpu
from jax.experimental.pallas import tpu_sc as plsc

N = 10000
E = 320000
D = 128
H = 64
NP = 10240

NC = 2
NS = 16
NW = NC * NS
EPW = E // NW
K = 80
NCHUNK = EPW // K
NB = 5
RPT = NP // NS
DEGW = H

_mesh = plsc.VectorSubcoreMesh(core_axis_name="c", subcore_axis_name="s")
_sc_params = pltpu.CompilerParams(use_tc_tiling_on_sc=False)

_INIT_NP = np.zeros((2 * RPT, DEGW), np.float32)
_INIT_NP[:RPT] = 1.0


@functools.partial(
    pl.kernel,
    out_type=(
        jax.ShapeDtypeStruct((NC * NP, DEGW), jnp.float32),
        jax.ShapeDtypeStruct((NW, NCHUNK, K), jnp.int32),
    ),
    mesh=_mesh,
    compiler_params=_sc_params,
    scratch_types=[
        pltpu.VMEM_SHARED((NP, DEGW), jnp.float32),
        pltpu.VMEM((NCHUNK, K), jnp.int32),
        pltpu.VMEM((K, DEGW), jnp.float32),
        pltpu.SemaphoreType.DMA,
        pltpu.SemaphoreType.DMA,
    ],
)
def _sc_degree(dst3d_in, init_hbm, deg_out, dst3d_out,
               deg_sh, dst_v, ones_v, sem0, sem1):
    c = lax.axis_index("c")
    s = lax.axis_index("s")
    wid = c * NS + s
    pltpu.sync_copy(init_hbm.at[pl.ds(c * RPT, RPT)], deg_sh.at[pl.ds(s * RPT, RPT)])
    pltpu.sync_copy(init_hbm.at[pl.ds(0, K)], ones_v)
    pltpu.sync_copy(dst3d_in.at[wid], dst_v)

    def s_start(m, sem):
        pltpu.async_copy(ones_v, deg_sh.at[dst_v.at[m]], sem, add=True)

    def s_wait(m, sem):
        pltpu.make_async_copy(ones_v, deg_sh.at[dst_v.at[m]], sem).wait()

    plsc.subcore_barrier()
    s_start(0, sem0)

    def body(j, carry):
        m1 = 2 * j + 1
        s_start(m1, sem1)
        s_wait(m1 - 1, sem0)
        s_start(m1 + 1, sem0)
        s_wait(m1, sem1)
        return carry

    lax.fori_loop(0, (NCHUNK - 1) // 2, body, 0)
    pltpu.sync_copy(dst_v, dst3d_out.at[wid])
    s_wait(NCHUNK - 1, sem0)
    plsc.subcore_barrier()
    pltpu.sync_copy(deg_sh.at[pl.ds(s * RPT, RPT)],
                    deg_out.at[pl.ds(c * NP + s * RPT, RPT)])


def _edge_body(g_hbm, src3d, dst3d, acc_out, src3d_out, acc_sh, g_sh, src_v,
               dst_v, rows_v, gsem, ssem):
    c = lax.axis_index("c")
    s = lax.axis_index("s")
    wid = c * NS + s
    pltpu.sync_copy(g_hbm.at[pl.ds(s * RPT, RPT)], acc_sh.at[pl.ds(s * RPT, RPT)])
    pltpu.sync_copy(g_hbm.at[pl.ds(s * RPT, RPT)], g_sh.at[pl.ds(s * RPT, RPT)])
    pltpu.sync_copy(src3d.at[wid], src_v)
    pltpu.sync_copy(dst3d.at[wid], dst_v)
    if src3d_out is not None:
        pltpu.sync_copy(src_v, src3d_out.at[wid])
    plsc.subcore_barrier()

    def g_src(b):
        return g_sh if b == 3 else g_hbm

    def g_start(q, b):
        pltpu.async_copy(g_src(b).at[src_v.at[q]], rows_v.at[b], gsem[b])

    def g_wait(q, b):
        pltpu.make_async_copy(g_src(b).at[src_v.at[q]], rows_v.at[b], gsem[b]).wait()

    def s_start(m, b):
        pltpu.async_copy(rows_v.at[b], acc_sh.at[dst_v.at[m]], ssem[b], add=True)

    def s_wait(m, b):
        pltpu.make_async_copy(rows_v.at[b], acc_sh.at[dst_v.at[m]], ssem[b]).wait()

    def step(m, b, do_wait_prev, gather_q):
        g_wait(m, b)
        s_start(m, b)
        if do_wait_prev:
            s_wait(m - 1, (b - 1) % NB)
        if gather_q:
            g_start(m + NB - 1, (b + NB - 1) % NB)

    for b in range(NB - 1):
        g_start(b, b)
    for b in range(NB):
        step(b, b, b > 0, True)

    def body(j, carry):
        m0 = NB * j
        for b in range(NB):
            step(m0 + b, b, True, True)
        return carry

    lax.fori_loop(1, NCHUNK // NB - 1, body, 0)

    m0 = NCHUNK - NB
    for b in range(NB):
        step(m0 + b, b, True, b == 0)
    s_wait(NCHUNK - 1, (NCHUNK - 1) % NB)
    plsc.subcore_barrier()
    pltpu.sync_copy(acc_sh.at[pl.ds(s * RPT, RPT)],
                    acc_out.at[pl.ds(c * NP + s * RPT, RPT)])


_EDGE_SCRATCH = [
    pltpu.VMEM_SHARED((NP, H), jnp.float32),
    pltpu.VMEM_SHARED((NP, H), jnp.float32),
    pltpu.VMEM((NCHUNK, K), jnp.int32),
    pltpu.VMEM((NCHUNK, K), jnp.int32),
    pltpu.VMEM((NB, K, H), jnp.float32),
    [pltpu.SemaphoreType.DMA] * NB,
    [pltpu.SemaphoreType.DMA] * NB,
]
_ACC_TYPE = jax.ShapeDtypeStruct((NC * NP, H), jnp.float32)
_IDX_TYPE = jax.ShapeDtypeStruct((NW, NCHUNK, K), jnp.int32)


@functools.partial(pl.kernel, out_type=(_ACC_TYPE, _IDX_TYPE), mesh=_mesh,
                   compiler_params=_sc_params, scratch_types=_EDGE_SCRATCH)
def _sc_edges1(g_hbm, src3d, dst3d, acc_out, src3d_out, acc_sh, g_sh, src_v,
               dst_v, rows_v, gsem, ssem):
    _edge_body(g_hbm, src3d, dst3d, acc_out, src3d_out, acc_sh, g_sh, src_v,
               dst_v, rows_v, gsem, ssem)


@functools.partial(pl.kernel, out_type=_ACC_TYPE, mesh=_mesh,
                   compiler_params=_sc_params, scratch_types=_EDGE_SCRATCH)
def _sc_edges2(g_hbm, src3d, dst3d, acc_out, acc_sh, g_sh, src_v, dst_v,
               rows_v, gsem, ssem):
    _edge_body(g_hbm, src3d, dst3d, acc_out, None, acc_sh, g_sh, src_v,
               dst_v, rows_v, gsem, ssem)


_PB = 512
_GRID = NP // (2 * _PB)
_HI = NP // (2 * _PB)


def _pk_spec(off=0):
    return pl.BlockSpec((_PB, 128), lambda i, o=off: (i + o, 0))


def _full_spec(r, c):
    return pl.BlockSpec((r, c), lambda i: (0, 0))


def _tc_mm_body(xp_b, w_b, o_b):
    o_b[...] = jnp.dot(xp_b[...], w_b[...], preferred_element_type=jnp.float32)


def _tc_scale_body(xw_b, d0_b, d1_b, o_b):
    dinv = lax.rsqrt(d0_b[...] + d1_b[...])
    o_b[...] = xw_b[...] * dinv


def _tc_mid_body(a0_b, a1_b, g_b, d0_b, d1_b, w_b, b_b, o_b):
    dinv = lax.rsqrt(d0_b[...] + d1_b[...])
    tot = a0_b[...] + a1_b[...] - g_b[...]
    h = jnp.maximum(tot * dinv + b_b[...], 0.0)
    o_b[...] = jnp.dot(h, w_b[...], preferred_element_type=jnp.float32) * dinv


def _tc_last_body(a0_b, a1_b, g_b, d0_b, d1_b, b_b, o_b):
    dinv = lax.rsqrt(d0_b[...] + d1_b[...])
    tot = a0_b[...] + a1_b[...] - g_b[...]
    o_b[...] = jnp.maximum(tot * dinv + b_b[...], 0.0)


def _tc_mm(xp, wbd):
    return pl.pallas_call(
        _tc_mm_body,
        grid=(_GRID,),
        in_specs=[pl.BlockSpec((_PB, 2 * D), lambda i: (i, 0)),
                  _full_spec(2 * D, 128)],
        out_specs=_pk_spec(),
        out_shape=jax.ShapeDtypeStruct((NP // 2, 128), jnp.float32),
    )(xp, wbd)


def _tc_scale(xwp, degp):
    return pl.pallas_call(
        _tc_scale_body,
        grid=(_GRID,),
        in_specs=[_pk_spec(), _pk_spec(), _pk_spec(_HI)],
        out_specs=_pk_spec(),
        out_shape=jax.ShapeDtypeStruct((NP // 2, 128), jnp.float32),
    )(xwp, degp, degp)


def _tc_mid(accp, gp, degp, wbd, bx):
    return pl.pallas_call(
        _tc_mid_body,
        grid=(_GRID,),
        in_specs=[_pk_spec(), _pk_spec(_HI), _pk_spec(),
                  _pk_spec(), _pk_spec(_HI),
                  _full_spec(128, 128), _full_spec(1, 128)],
        out_specs=_pk_spec(),
        out_shape=jax.ShapeDtypeStruct((NP // 2, 128), jnp.float32),
    )(accp, accp, gp, degp, degp, wbd, bx)


def _tc_last(accp, gp, degp, bx):
    return pl.pallas_call(
        _tc_last_body,
        grid=(_GRID,),
        in_specs=[_pk_spec(), _pk_spec(_HI), _pk_spec(),
                  _pk_spec(), _pk_spec(_HI), _full_spec(1, 128)],
        out_specs=_pk_spec(),
        out_shape=jax.ShapeDtypeStruct((N // 2, 128), jnp.float32),
    )(accp, accp, gp, degp, degp, bx)


def _blockdiag(w):
    k, m = w.shape
    return jnp.zeros((2 * k, 2 * m), w.dtype).at[:k, :m].set(w).at[k:, m:].set(w)


@jax.jit
def kernel(x, edge_index, W1, b1, W2, b2):
    src3d_in = edge_index[0].reshape(NW, NCHUNK, K)
    dst3d_in = edge_index[1].reshape(NW, NCHUNK, K)
    init_const = jnp.asarray(_INIT_NP)
    xp = x.reshape(N // 2, 2 * D)
    w1bd = _blockdiag(W1)
    w2bd = _blockdiag(W2)
    b1x = jnp.concatenate([b1, b1]).reshape(1, 128)
    b2x = jnp.concatenate([b2, b2]).reshape(1, 128)

    deg2, dst3d = _sc_degree(dst3d_in, init_const)
    degp = deg2.reshape(NC * NP // 2, 128)
    xwp = _tc_mm(xp, w1bd)
    g1p = _tc_scale(xwp, degp)
    g1 = g1p.reshape(NP, H)
    acc1, src3d = _sc_edges1(g1, src3d_in, dst3d)
    g2p = _tc_mid(acc1.reshape(NC * NP // 2, 128), g1p, degp, w2bd, b1x)
    acc2 = _sc_edges2(g2p.reshape(NP, H), src3d, dst3d)
    outp = _tc_last(acc2.reshape(NC * NP // 2, 128), g2p, degp, b2x)
    return outp.reshape(N, H)


# speedup vs baseline: 1.2279x; 1.2279x over previous
import functools

import numpy as np
import jax
import jax.numpy as jnp
from jax import lax
from jax.experimental import pallas as pl
from jax.experimental.pallas import tpu as pltpu
from jax.experimental.pallas import tpu_sc as plsc

N = 10000
E = 320000
D = 128
H = 64
NP = 10240

NC = 2
NS = 16
NW = NC * NS
EPW = E // NW
K = 80
NCHUNK = EPW // K
NB = 5
RPT = NP // NS
DEGW = H

_mesh = plsc.VectorSubcoreMesh(core_axis_name="c", subcore_axis_name="s")
_sc_params = pltpu.CompilerParams(use_tc_tiling_on_sc=False)

_INIT_NP = np.zeros((2 * RPT, DEGW), np.float32)
_INIT_NP[:RPT] = 1.0


@functools.partial(
    pl.kernel,
    out_type=jax.ShapeDtypeStruct((NC * NP, DEGW), jnp.float32),
    mesh=_mesh,
    compiler_params=_sc_params,
    scratch_types=[
        pltpu.VMEM_SHARED((NP, DEGW), jnp.float32),
        pltpu.VMEM((NCHUNK, K), jnp.int32),
        pltpu.VMEM((K, DEGW), jnp.float32),
        pltpu.SemaphoreType.DMA,
        pltpu.SemaphoreType.DMA,
    ],
)
def _sc_degree(ei, init_hbm, deg_out, deg_sh, dst_v, ones_v, sem0, sem1):
    c = lax.axis_index("c")
    s = lax.axis_index("s")
    wid = c * NS + s
    pltpu.sync_copy(init_hbm.at[pl.ds(c * RPT, RPT)], deg_sh.at[pl.ds(s * RPT, RPT)])
    pltpu.sync_copy(init_hbm.at[pl.ds(0, K)], ones_v)
    pltpu.sync_copy(ei.at[1, wid], dst_v)

    def s_start(m, sem):
        pltpu.async_copy(ones_v, deg_sh.at[dst_v.at[m]], sem, add=True)

    def s_wait(m, sem):
        pltpu.make_async_copy(ones_v, deg_sh.at[dst_v.at[m]], sem).wait()

    plsc.subcore_barrier()
    s_start(0, sem0)

    def body(j, carry):
        m1 = 2 * j + 1
        s_start(m1, sem1)
        s_wait(m1 - 1, sem0)
        s_start(m1 + 1, sem0)
        s_wait(m1, sem1)
        return carry

    lax.fori_loop(0, (NCHUNK - 1) // 2, body, 0)
    s_wait(NCHUNK - 1, sem0)
    plsc.subcore_barrier()
    pltpu.sync_copy(deg_sh.at[pl.ds(s * RPT, RPT)],
                    deg_out.at[pl.ds(c * NP + s * RPT, RPT)])


def _edge_body(g_hbm, ei, acc_out, acc_sh, src_v, dst_v, rows_v, gsem, ssem):
    c = lax.axis_index("c")
    s = lax.axis_index("s")
    wid = c * NS + s
    pltpu.sync_copy(g_hbm.at[pl.ds(s * RPT, RPT)], acc_sh.at[pl.ds(s * RPT, RPT)])
    pltpu.sync_copy(ei.at[0, wid], src_v)
    pltpu.sync_copy(ei.at[1, wid], dst_v)
    plsc.subcore_barrier()

    def g_start(q, b):
        pltpu.async_copy(g_hbm.at[src_v.at[q]], rows_v.at[b], gsem[b])

    def g_wait(q, b):
        pltpu.make_async_copy(g_hbm.at[src_v.at[q]], rows_v.at[b], gsem[b]).wait()

    def s_start(m, b):
        pltpu.async_copy(rows_v.at[b], acc_sh.at[dst_v.at[m]], ssem[b], add=True)

    def s_wait(m, b):
        pltpu.make_async_copy(rows_v.at[b], acc_sh.at[dst_v.at[m]], ssem[b]).wait()

    def step(m, b, do_wait_prev, gather_q):
        g_wait(m, b)
        s_start(m, b)
        if do_wait_prev:
            s_wait(m - 1, (b - 1) % NB)
        if gather_q:
            g_start(m + NB - 1, (b + NB - 1) % NB)

    for b in range(NB - 1):
        g_start(b, b)
    for b in range(NB):
        step(b, b, b > 0, True)

    def body(j, carry):
        m0 = NB * j
        for b in range(NB):
            step(m0 + b, b, True, True)
        return carry

    lax.fori_loop(1, NCHUNK // NB - 1, body, 0)

    m0 = NCHUNK - NB
    for b in range(NB):
        step(m0 + b, b, True, b == 0)
    s_wait(NCHUNK - 1, (NCHUNK - 1) % NB)
    plsc.subcore_barrier()
    pltpu.sync_copy(acc_sh.at[pl.ds(s * RPT, RPT)],
                    acc_out.at[pl.ds(c * NP + s * RPT, RPT)])


_EDGE_SCRATCH = [
    pltpu.VMEM_SHARED((NP, H), jnp.float32),
    pltpu.VMEM((NCHUNK, K), jnp.int32),
    pltpu.VMEM((NCHUNK, K), jnp.int32),
    pltpu.VMEM((NB, K, H), jnp.float32),
    [pltpu.SemaphoreType.DMA] * NB,
    [pltpu.SemaphoreType.DMA] * NB,
]
_ACC_TYPE = jax.ShapeDtypeStruct((NC * NP, H), jnp.float32)


@functools.partial(pl.kernel, out_type=_ACC_TYPE, mesh=_mesh,
                   compiler_params=_sc_params, scratch_types=_EDGE_SCRATCH)
def _sc_edges(g_hbm, ei, acc_out, acc_sh, src_v, dst_v, rows_v, gsem, ssem):
    _edge_body(g_hbm, ei, acc_out, acc_sh, src_v, dst_v, rows_v, gsem, ssem)


_PB = 512
_GRID = NP // (2 * _PB)
_HI = NP // (2 * _PB)


def _pk_spec(off=0):
    return pl.BlockSpec((_PB, 128), lambda i, o=off: (i + o, 0))


def _full_spec(r, c):
    return pl.BlockSpec((r, c), lambda i: (0, 0))


def _tc_mm_body(xp_b, w_b, o_b):
    o_b[...] = jnp.dot(xp_b[...], w_b[...], preferred_element_type=jnp.float32)


def _tc_scale_body(xw_b, d0_b, d1_b, o_b):
    dinv = lax.rsqrt(d0_b[...] + d1_b[...])
    o_b[...] = xw_b[...] * dinv


def _tc_mid_body(a0_b, a1_b, g_b, d0_b, d1_b, w_b, b_b, o_b):
    dinv = lax.rsqrt(d0_b[...] + d1_b[...])
    tot = a0_b[...] + a1_b[...] - g_b[...]
    h = jnp.maximum(tot * dinv + b_b[...], 0.0)
    o_b[...] = jnp.dot(h, w_b[...], preferred_element_type=jnp.float32) * dinv


def _tc_last_body(a0_b, a1_b, g_b, d0_b, d1_b, b_b, o_b):
    dinv = lax.rsqrt(d0_b[...] + d1_b[...])
    tot = a0_b[...] + a1_b[...] - g_b[...]
    o_b[...] = jnp.maximum(tot * dinv + b_b[...], 0.0)


def _tc_mm(xp, wbd):
    return pl.pallas_call(
        _tc_mm_body,
        grid=(_GRID,),
        in_specs=[pl.BlockSpec((_PB, 2 * D), lambda i: (i, 0)),
                  _full_spec(2 * D, 128)],
        out_specs=_pk_spec(),
        out_shape=jax.ShapeDtypeStruct((NP // 2, 128), jnp.float32),
    )(xp, wbd)


def _tc_scale(xwp, degp):
    return pl.pallas_call(
        _tc_scale_body,
        grid=(_GRID,),
        in_specs=[_pk_spec(), _pk_spec(), _pk_spec(_HI)],
        out_specs=_pk_spec(),
        out_shape=jax.ShapeDtypeStruct((NP // 2, 128), jnp.float32),
    )(xwp, degp, degp)


def _tc_mid(accp, gp, degp, wbd, bx):
    return pl.pallas_call(
        _tc_mid_body,
        grid=(_GRID,),
        in_specs=[_pk_spec(), _pk_spec(_HI), _pk_spec(),
                  _pk_spec(), _pk_spec(_HI),
                  _full_spec(128, 128), _full_spec(1, 128)],
        out_specs=_pk_spec(),
        out_shape=jax.ShapeDtypeStruct((NP // 2, 128), jnp.float32),
    )(accp, accp, gp, degp, degp, wbd, bx)


def _tc_last(accp, gp, degp, bx):
    return pl.pallas_call(
        _tc_last_body,
        grid=(_GRID,),
        in_specs=[_pk_spec(), _pk_spec(_HI), _pk_spec(),
                  _pk_spec(), _pk_spec(_HI), _full_spec(1, 128)],
        out_specs=_pk_spec(),
        out_shape=jax.ShapeDtypeStruct((N // 2, 128), jnp.float32),
    )(accp, accp, gp, degp, degp, bx)


def _blockdiag(w):
    k, m = w.shape
    return jnp.zeros((2 * k, 2 * m), w.dtype).at[:k, :m].set(w).at[k:, m:].set(w)


@jax.jit
def kernel(x, edge_index, W1, b1, W2, b2):
    ei = edge_index.reshape(2, NW, NCHUNK, K)
    init_const = jnp.asarray(_INIT_NP)
    xp = x.reshape(N // 2, 2 * D)
    w1bd = _blockdiag(W1)
    w2bd = _blockdiag(W2)
    b1x = jnp.concatenate([b1, b1]).reshape(1, 128)
    b2x = jnp.concatenate([b2, b2]).reshape(1, 128)

    deg2 = _sc_degree(ei, init_const)
    degp = deg2.reshape(NC * NP // 2, 128)
    xwp = _tc_mm(xp, w1bd)
    g1p = _tc_scale(xwp, degp)
    g1 = g1p.reshape(NP, H)
    acc1 = _sc_edges(g1, ei)
    g2p = _tc_mid(acc1.reshape(NC * NP // 2, 128), g1p, degp, w2bd, b1x)
    acc2 = _sc_edges(g2p.reshape(NP, H), ei)
    outp = _tc_last(acc2.reshape(NC * NP // 2, 128), g2p, degp, b2x)
    return outp.reshape(N, H)


# speedup vs baseline: 1.3518x; 1.1010x over previous
import functools

import numpy as np
import jax
import jax.numpy as jnp
from jax import lax
from jax.experimental import pallas as pl
from jax.experimental.pallas import tpu as pltpu
from jax.experimental.pallas import tpu_sc as plsc

N = 10000
E = 320000
D = 128
H = 64
NP = 10240

NC = 2
NS = 16
NW = NC * NS
EPW = E // NW
K = 80
NCHUNK = EPW // K
NB = 5
RPT = NP // NS
DEGW = 16

_mesh = plsc.VectorSubcoreMesh(core_axis_name="c", subcore_axis_name="s")
_sc_params = pltpu.CompilerParams(use_tc_tiling_on_sc=False)

_INIT_NP = np.zeros((2 * RPT, DEGW), np.float32)
_INIT_NP[:RPT] = 1.0


@functools.partial(
    pl.kernel,
    out_type=jax.ShapeDtypeStruct((NC * NP, H), jnp.float32),
    mesh=_mesh,
    compiler_params=_sc_params,
    scratch_types=[
        pltpu.VMEM_SHARED((NP, DEGW), jnp.float32),
        pltpu.VMEM((NCHUNK, K), jnp.int32),
        pltpu.VMEM((K, DEGW), jnp.float32),
        pltpu.VMEM((RPT, DEGW), jnp.float32),
        pltpu.VMEM((RPT, H), jnp.float32),
        pltpu.SemaphoreType.DMA,
        pltpu.SemaphoreType.DMA,
    ],
)
def _sc_degree(ei, init_hbm, deg_out, deg_sh, dst_v, ones_v, d16_v, d64_v,
               sem0, sem1):
    c = lax.axis_index("c")
    s = lax.axis_index("s")
    wid = c * NS + s
    pltpu.sync_copy(init_hbm.at[pl.ds(c * RPT, RPT)], deg_sh.at[pl.ds(s * RPT, RPT)])
    pltpu.sync_copy(init_hbm.at[pl.ds(0, K)], ones_v)
    pltpu.sync_copy(ei.at[1, wid], dst_v)

    def s_start(m, sem):
        pltpu.async_copy(ones_v, deg_sh.at[dst_v.at[m]], sem, add=True)

    def s_wait(m, sem):
        pltpu.make_async_copy(ones_v, deg_sh.at[dst_v.at[m]], sem).wait()

    plsc.subcore_barrier()
    s_start(0, sem0)

    def body(j, carry):
        m1 = 2 * j + 1
        s_start(m1, sem1)
        s_wait(m1 - 1, sem0)
        s_start(m1 + 1, sem0)
        s_wait(m1, sem1)
        return carry

    lax.fori_loop(0, (NCHUNK - 1) // 2, body, 0)
    s_wait(NCHUNK - 1, sem0)
    plsc.subcore_barrier()
    pltpu.sync_copy(deg_sh.at[pl.ds(s * RPT, RPT)], d16_v)

    def expand(j, carry):
        r0 = 8 * j
        for u in range(8):
            v = d16_v[r0 + u, :]
            for h4 in range(H // DEGW):
                d64_v[r0 + u, pl.ds(h4 * DEGW, DEGW)] = v
        return carry

    lax.fori_loop(0, RPT // 8, expand, 0)
    pltpu.sync_copy(d64_v, deg_out.at[pl.ds(c * NP + s * RPT, RPT)])


def _edge_body(g_hbm, ei, acc_out, acc_sh, src_v, dst_v, rows_v, gsem, ssem):
    c = lax.axis_index("c")
    s = lax.axis_index("s")
    wid = c * NS + s
    pltpu.sync_copy(g_hbm.at[pl.ds(s * RPT, RPT)], acc_sh.at[pl.ds(s * RPT, RPT)])
    pltpu.sync_copy(ei.at[0, wid], src_v)
    pltpu.sync_copy(ei.at[1, wid], dst_v)
    plsc.subcore_barrier()

    def g_start(q, b):
        pltpu.async_copy(g_hbm.at[src_v.at[q]], rows_v.at[b], gsem[b])

    def g_wait(q, b):
        pltpu.make_async_copy(g_hbm.at[src_v.at[q]], rows_v.at[b], gsem[b]).wait()

    def s_start(m, b):
        pltpu.async_copy(rows_v.at[b], acc_sh.at[dst_v.at[m]], ssem[b], add=True)

    def s_wait(m, b):
        pltpu.make_async_copy(rows_v.at[b], acc_sh.at[dst_v.at[m]], ssem[b]).wait()

    def step(m, b, do_wait_prev, gather_q):
        g_wait(m, b)
        s_start(m, b)
        if do_wait_prev:
            s_wait(m - 1, (b - 1) % NB)
        if gather_q:
            g_start(m + NB - 1, (b + NB - 1) % NB)

    for b in range(NB - 1):
        g_start(b, b)
    for b in range(NB):
        step(b, b, b > 0, True)

    def body(j, carry):
        m0 = NB * j
        for b in range(NB):
            step(m0 + b, b, True, True)
        return carry

    lax.fori_loop(1, NCHUNK // NB - 1, body, 0)

    m0 = NCHUNK - NB
    for b in range(NB):
        step(m0 + b, b, True, b == 0)
    s_wait(NCHUNK - 1, (NCHUNK - 1) % NB)
    plsc.subcore_barrier()
    pltpu.sync_copy(acc_sh.at[pl.ds(s * RPT, RPT)],
                    acc_out.at[pl.ds(c * NP + s * RPT, RPT)])


_EDGE_SCRATCH = [
    pltpu.VMEM_SHARED((NP, H), jnp.float32),
    pltpu.VMEM((NCHUNK, K), jnp.int32),
    pltpu.VMEM((NCHUNK, K), jnp.int32),
    pltpu.VMEM((NB, K, H), jnp.float32),
    [pltpu.SemaphoreType.DMA] * NB,
    [pltpu.SemaphoreType.DMA] * NB,
]
_ACC_TYPE = jax.ShapeDtypeStruct((NC * NP, H), jnp.float32)


@functools.partial(pl.kernel, out_type=_ACC_TYPE, mesh=_mesh,
                   compiler_params=_sc_params, scratch_types=_EDGE_SCRATCH)
def _sc_edges(g_hbm, ei, acc_out, acc_sh, src_v, dst_v, rows_v, gsem, ssem):
    _edge_body(g_hbm, ei, acc_out, acc_sh, src_v, dst_v, rows_v, gsem, ssem)


_PB = 512
_GRID = NP // (2 * _PB)
_HI = NP // (2 * _PB)


def _pk_spec(off=0):
    return pl.BlockSpec((_PB, 128), lambda i, o=off: (i + o, 0))


def _full_spec(r, c):
    return pl.BlockSpec((r, c), lambda i: (0, 0))


def _tc_mm_body(xp_b, w_b, o_b):
    o_b[...] = jnp.dot(xp_b[...], w_b[...], preferred_element_type=jnp.float32)


def _tc_scale_body(xw_b, d0_b, d1_b, o_b):
    dinv = lax.rsqrt(d0_b[...] + d1_b[...])
    o_b[...] = xw_b[...] * dinv


def _tc_mid_body(a0_b, a1_b, g_b, d0_b, d1_b, w_b, b_b, o_b):
    dinv = lax.rsqrt(d0_b[...] + d1_b[...])
    tot = a0_b[...] + a1_b[...] - g_b[...]
    h = jnp.maximum(tot * dinv + b_b[...], 0.0)
    o_b[...] = jnp.dot(h, w_b[...], preferred_element_type=jnp.float32) * dinv


def _tc_last_body(a0_b, a1_b, g_b, d0_b, d1_b, b_b, o_b):
    dinv = lax.rsqrt(d0_b[...] + d1_b[...])
    tot = a0_b[...] + a1_b[...] - g_b[...]
    o_b[...] = jnp.maximum(tot * dinv + b_b[...], 0.0)


def _tc_mm(xp, wbd):
    return pl.pallas_call(
        _tc_mm_body,
        grid=(_GRID,),
        in_specs=[pl.BlockSpec((_PB, 2 * D), lambda i: (i, 0)),
                  _full_spec(2 * D, 128)],
        out_specs=_pk_spec(),
        out_shape=jax.ShapeDtypeStruct((NP // 2, 128), jnp.float32),
    )(xp, wbd)


def _tc_scale(xwp, degp):
    return pl.pallas_call(
        _tc_scale_body,
        grid=(_GRID,),
        in_specs=[_pk_spec(), _pk_spec(), _pk_spec(_HI)],
        out_specs=_pk_spec(),
        out_shape=jax.ShapeDtypeStruct((NP // 2, 128), jnp.float32),
    )(xwp, degp, degp)


def _tc_mid(accp, gp, degp, wbd, bx):
    return pl.pallas_call(
        _tc_mid_body,
        grid=(_GRID,),
        in_specs=[_pk_spec(), _pk_spec(_HI), _pk_spec(),
                  _pk_spec(), _pk_spec(_HI),
                  _full_spec(128, 128), _full_spec(1, 128)],
        out_specs=_pk_spec(),
        out_shape=jax.ShapeDtypeStruct((NP // 2, 128), jnp.float32),
    )(accp, accp, gp, degp, degp, wbd, bx)


def _tc_last(accp, gp, degp, bx):
    return pl.pallas_call(
        _tc_last_body,
        grid=(_GRID,),
        in_specs=[_pk_spec(), _pk_spec(_HI), _pk_spec(),
                  _pk_spec(), _pk_spec(_HI), _full_spec(1, 128)],
        out_specs=_pk_spec(),
        out_shape=jax.ShapeDtypeStruct((N // 2, 128), jnp.float32),
    )(accp, accp, gp, degp, degp, bx)


def _blockdiag(w):
    k, m = w.shape
    return jnp.zeros((2 * k, 2 * m), w.dtype).at[:k, :m].set(w).at[k:, m:].set(w)


@jax.jit
def kernel(x, edge_index, W1, b1, W2, b2):
    ei = edge_index.reshape(2, NW, NCHUNK, K)
    init_const = jnp.asarray(_INIT_NP)
    xp = x.reshape(N // 2, 2 * D)
    w1bd = _blockdiag(W1)
    w2bd = _blockdiag(W2)
    b1x = jnp.concatenate([b1, b1]).reshape(1, 128)
    b2x = jnp.concatenate([b2, b2]).reshape(1, 128)

    deg2 = _sc_degree(ei, init_const)
    degp = deg2.reshape(NC * NP // 2, 128)
    xwp = _tc_mm(xp, w1bd)
    g1p = _tc_scale(xwp, degp)
    g1 = g1p.reshape(NP, H)
    acc1 = _sc_edges(g1, ei)
    g2p = _tc_mid(acc1.reshape(NC * NP // 2, 128), g1p, degp, w2bd, b1x)
    acc2 = _sc_edges(g2p.reshape(NP, H), ei)
    outp = _tc_last(acc2.reshape(NC * NP // 2, 128), g2p, degp, b2x)
    return outp.reshape(N, H)
